# baseline (speedup 1.0000x reference)
import functools

import jax
import jax.numpy as jnp
from jax import lax
from jax.experimental import pallas as pl
from jax.experimental.pallas import tpu as pltpu
from jax.experimental.pallas import tpu_sc as plsc

BATCH = 16384
DIM = 128
HIDDEN = 64
NUM_CORES = 2
NUM_SUBCORES = 16
NUM_WORKERS = NUM_CORES * NUM_SUBCORES
B_PER_W = BATCH // NUM_WORKERS
CHUNK = 128
NBUF = 7
NCHUNKS_PER_TABLE = B_PER_W // CHUNK
NCHUNKS = 2 * NCHUNKS_PER_TABLE


def _sc_gather(emb_p, emb_n, idx_p, idx_n):
  mesh = plsc.VectorSubcoreMesh(core_axis_name="c", subcore_axis_name="s")

  @functools.partial(
      pl.kernel,
      mesh=mesh,
      out_type=jax.ShapeDtypeStruct((BATCH, 2 * DIM), jnp.float32),
      scratch_types=[
          pltpu.VMEM((B_PER_W,), jnp.int32),
          pltpu.VMEM((B_PER_W,), jnp.int32),
          pltpu.VMEM((NBUF, CHUNK, DIM), jnp.float32),
          pltpu.SemaphoreType.DMA((NBUF,)),
          pltpu.SemaphoreType.DMA((NBUF,)),
      ],
  )
  def gather_kernel(embp_hbm, embn_hbm, idxp_hbm, idxn_hbm,
                    out_hbm, idxp_v, idxn_v, rows_v, gsem, wsem):
    wid = lax.axis_index("s") * NUM_CORES + lax.axis_index("c")
    base = wid * B_PER_W
    cp0 = pltpu.make_async_copy(
        idxp_hbm.at[pl.ds(base, B_PER_W)], idxp_v, gsem.at[0])
    cp1 = pltpu.make_async_copy(
        idxn_hbm.at[pl.ds(base, B_PER_W)], idxn_v, gsem.at[1])
    cp0.start()
    cp1.start()
    cp0.wait()
    cp1.wait()

    sched = []
    for k in range(NCHUNKS_PER_TABLE):
      sched.append((idxp_v, embp_hbm, 0, k * CHUNK))
    for k in range(NCHUNKS_PER_TABLE):
      sched.append((idxn_v, embn_hbm, DIM, k * CHUNK))

    def g_start(c):
      idx_v, emb, _, off = sched[c]
      b = c % NBUF
      return pltpu.async_copy(
          emb.at[idx_v.at[pl.ds(off, CHUNK)]], rows_v.at[b], gsem.at[b])

    def w_start(c):
      _, _, col, off = sched[c]
      b = c % NBUF
      return pltpu.async_copy(
          rows_v.at[b],
          out_hbm.at[pl.ds(base + off, CHUNK), pl.ds(col, DIM)], wsem.at[b])

    gathers = [None] * NCHUNKS
    writes = [None] * NCHUNKS
    for c in range(min(NBUF, NCHUNKS)):
      gathers[c] = g_start(c)
    for c in range(NCHUNKS):
      gathers[c].wait()
      writes[c] = w_start(c)
      nxt = c + NBUF
      if nxt < NCHUNKS:
        writes[c].wait()
        gathers[nxt] = g_start(nxt)
    for c in range(max(0, NCHUNKS - NBUF), NCHUNKS):
      writes[c].wait()

  return gather_kernel(emb_p, emb_n, idx_p, idx_n)


def _tc_mlp(rows, w1, b1_row, w2, b2_11):
  bm = 8192
  grid = (BATCH // bm,)
  dn = (((1,), (1,)), ((), ()))

  def body(r_ref, w1_ref, b1_ref, w2_ref, b2_ref, o_ref):
    h = lax.dot_general(r_ref[...], w1_ref[...], dn,
                        preferred_element_type=jnp.float32)
    h = jnp.maximum(h + b1_ref[...], 0.0)
    o_ref[...] = jnp.sum(h * w2_ref[...], axis=1, keepdims=True) + b2_ref[0, 0]

  return pl.pallas_call(
      body,
      grid=grid,
      in_specs=[
          pl.BlockSpec((bm, 2 * DIM), lambda i: (i, 0)),
          pl.BlockSpec((HIDDEN, 2 * DIM), lambda i: (0, 0)),
          pl.BlockSpec((1, HIDDEN), lambda i: (0, 0)),
          pl.BlockSpec((1, HIDDEN), lambda i: (0, 0)),
          pl.BlockSpec((1, 1), lambda i: (0, 0)),
      ],
      out_specs=pl.BlockSpec((bm, 1), lambda i: (i, 0)),
      out_shape=jax.ShapeDtypeStruct((BATCH, 1), jnp.float32),
      compiler_params=pltpu.CompilerParams(
          dimension_semantics=("parallel",)),
  )(rows, w1, b1_row, w2, b2_11)


@jax.jit
def kernel(x, emb_proton, emb_neutron, W1, b1, W2, b2):
  idx = x.astype(jnp.int32)
  idx_p = idx[:, 0]
  idx_n = idx[:, 1]
  rows = _sc_gather(emb_proton, emb_neutron, idx_p, idx_n)
  b1_row = b1.reshape(1, HIDDEN)
  b2_11 = b2.reshape(1, 1)
  return _tc_mlp(rows, W1, b1_row, W2, b2_11)
